# baseline (speedup 1.0000x reference)
import jax
import jax.numpy as jnp
from jax import lax
from jax.experimental import pallas as pl
from jax.experimental.pallas import tpu as pltpu
from jax.experimental.pallas import tpu_sc as plsc

BATCH = 4
SEQ_LEN = 8192
DIM = 1024
NC = 2
NS = 16
NW = NC * NS

EMB_TOTAL = SEQ_LEN * DIM
IN_TOTAL = BATCH * EMB_TOTAL
EW = EMB_TOTAL // NW
CH = 16384
NCH = EW // CH


def _sc_body(in_hbm, emb_hbm, out_hbm, emb_buf, io_buf):
    wid = lax.axis_index("s") * NC + lax.axis_index("c")
    base = wid * EW

    @pl.loop(0, NCH)
    def _chunk(c):
        emb_off = base + c * CH
        pltpu.sync_copy(emb_hbm.at[pl.ds(emb_off, CH)], emb_buf)
        for b in range(BATCH):
            off = b * EMB_TOTAL + emb_off
            pltpu.sync_copy(in_hbm.at[pl.ds(off, CH)], io_buf)

            @plsc.parallel_loop(0, CH, step=16, unroll=8)
            def _add(i):
                io_buf[pl.ds(i, 16)] += emb_buf[pl.ds(i, 16)]

            pltpu.sync_copy(io_buf, out_hbm.at[pl.ds(off, CH)])


def kernel(inputs, embeddings):
    in_flat = inputs.reshape(IN_TOTAL)
    emb_flat = embeddings.reshape(EMB_TOTAL)
    mesh = plsc.VectorSubcoreMesh(core_axis_name="c", subcore_axis_name="s")
    out = pl.kernel(
        _sc_body,
        out_type=jax.ShapeDtypeStruct((IN_TOTAL,), jnp.float32),
        mesh=mesh,
        scratch_types=[
            pltpu.VMEM((CH,), jnp.float32),
            pltpu.VMEM((CH,), jnp.float32),
        ],
    )(in_flat, emb_flat)
    return out.reshape(BATCH, SEQ_LEN, DIM)


# speedup vs baseline: 1.1126x; 1.1126x over previous
import jax
import jax.numpy as jnp
from jax import lax
from jax.experimental import pallas as pl
from jax.experimental.pallas import tpu as pltpu
from jax.experimental.pallas import tpu_sc as plsc

BATCH = 4
SEQ_LEN = 8192
DIM = 1024
NC = 2
NS = 16
NW = NC * NS

EMB_TOTAL = SEQ_LEN * DIM
IN_TOTAL = BATCH * EMB_TOTAL
EW = EMB_TOTAL // NW
CH = 16384
NCH = EW // CH
NSTEP = NCH * BATCH


def _in_off(base, k):
    c, b = k // BATCH, k % BATCH
    return b * EMB_TOTAL + base + c * CH


def _sc_body(in_hbm, emb_hbm, out_hbm,
             emb_buf, in_buf, out_buf,
             sem_emb, sem_in, sem_out):
    wid = lax.axis_index("s") * NC + lax.axis_index("c")
    base = wid * EW

    def in_cp(k, p):
        return pltpu.make_async_copy(
            in_hbm.at[pl.ds(_in_off(base, k), CH)], in_buf.at[p], sem_in.at[p])

    def out_cp(k, p):
        return pltpu.make_async_copy(
            out_buf.at[p], out_hbm.at[pl.ds(_in_off(base, k), CH)], sem_out.at[p])

    def emb_cp(c, q):
        return pltpu.make_async_copy(
            emb_hbm.at[pl.ds(base + c * CH, CH)], emb_buf.at[q], sem_emb.at[q])

    emb_cp(0, 0).start()
    in_cp(0, 0).start()
    in_cp(1, 1).start()

    for k in range(NSTEP):
        p = k % 2
        c, b = k // BATCH, k % BATCH
        q = c % 2
        if b == 0:
            emb_cp(c, q).wait()
            if c + 1 < NCH:
                emb_cp(c + 1, 1 - q).start()
        in_cp(k, p).wait()
        if k >= 2:
            out_cp(k - 2, p).wait()

        @plsc.parallel_loop(0, CH, step=16, unroll=8)
        def _add(i):
            out_buf[p, pl.ds(i, 16)] = (
                in_buf[p, pl.ds(i, 16)] + emb_buf[q, pl.ds(i, 16)])

        out_cp(k, p).start()
        if k + 2 < NSTEP:
            in_cp(k + 2, p).start()

    out_cp(NSTEP - 2, 0).wait()
    out_cp(NSTEP - 1, 1).wait()


def kernel(inputs, embeddings):
    in_flat = inputs.reshape(IN_TOTAL)
    emb_flat = embeddings.reshape(EMB_TOTAL)
    mesh = plsc.VectorSubcoreMesh(core_axis_name="c", subcore_axis_name="s")
    out = pl.kernel(
        _sc_body,
        out_type=jax.ShapeDtypeStruct((IN_TOTAL,), jnp.float32),
        mesh=mesh,
        scratch_types=[
            pltpu.VMEM((2, CH), jnp.float32),
            pltpu.VMEM((2, CH), jnp.float32),
            pltpu.VMEM((2, CH), jnp.float32),
            pltpu.SemaphoreType.DMA((2,)),
            pltpu.SemaphoreType.DMA((2,)),
            pltpu.SemaphoreType.DMA((2,)),
        ],
    )(in_flat, emb_flat)
    return out.reshape(BATCH, SEQ_LEN, DIM)


# speedup vs baseline: 3.6974x; 3.3232x over previous
import jax
import jax.numpy as jnp
from jax import lax
from jax.experimental import pallas as pl
from jax.experimental.pallas import tpu as pltpu
from jax.experimental.pallas import tpu_sc as plsc

BATCH = 4
SEQ_LEN = 8192
DIM = 1024
NC = 2
NS = 16
NW = NC * NS

EW = SEQ_LEN // NW
CR = 16
CH = CR * DIM
NCH = EW // CR
NSTEP = NCH * BATCH


def _row_off(base, k):
    c, b = k // BATCH, k % BATCH
    return b * SEQ_LEN + base + c * CR


def _sc_body(in_hbm, emb_hbm, out_hbm,
             emb_buf, in_buf, out_buf,
             sem_emb, sem_in, sem_out):
    wid = lax.axis_index("s") * NC + lax.axis_index("c")
    base = wid * EW

    def in_cp(k, p):
        return pltpu.make_async_copy(
            in_hbm.at[pl.ds(_row_off(base, k), CR)], in_buf.at[p], sem_in.at[p])

    def out_cp(k, p):
        return pltpu.make_async_copy(
            out_buf.at[p], out_hbm.at[pl.ds(_row_off(base, k), CR)], sem_out.at[p])

    def emb_cp(c, q):
        return pltpu.make_async_copy(
            emb_hbm.at[pl.ds(base + c * CR, CR)], emb_buf.at[q], sem_emb.at[q])

    emb_cp(0, 0).start()
    in_cp(0, 0).start()
    in_cp(1, 1).start()

    for k in range(NSTEP):
        p = k % 2
        c, b = k // BATCH, k % BATCH
        q = c % 2
        if b == 0:
            emb_cp(c, q).wait()
            if c + 1 < NCH:
                emb_cp(c + 1, 1 - q).start()
        in_cp(k, p).wait()
        if k >= 2:
            out_cp(k - 2, p).wait()

        @plsc.parallel_loop(0, CH, step=16, unroll=8)
        def _add(i):
            r = i >> 10
            j = pl.multiple_of(i & (DIM - 1), 16)
            out_buf[p, r, pl.ds(j, 16)] = (
                in_buf[p, r, pl.ds(j, 16)] + emb_buf[q, r, pl.ds(j, 16)])

        out_cp(k, p).start()
        if k + 2 < NSTEP:
            in_cp(k + 2, p).start()

    out_cp(NSTEP - 2, 0).wait()
    out_cp(NSTEP - 1, 1).wait()


def kernel(inputs, embeddings):
    in2d = inputs.reshape(BATCH * SEQ_LEN, DIM)
    mesh = plsc.VectorSubcoreMesh(core_axis_name="c", subcore_axis_name="s")
    out = pl.kernel(
        _sc_body,
        out_type=jax.ShapeDtypeStruct((BATCH * SEQ_LEN, DIM), jnp.float32),
        mesh=mesh,
        scratch_types=[
            pltpu.VMEM((2, CR, DIM), jnp.float32),
            pltpu.VMEM((2, CR, DIM), jnp.float32),
            pltpu.VMEM((2, CR, DIM), jnp.float32),
            pltpu.SemaphoreType.DMA((2,)),
            pltpu.SemaphoreType.DMA((2,)),
            pltpu.SemaphoreType.DMA((2,)),
        ],
    )(in2d, embeddings)
    return out.reshape(BATCH, SEQ_LEN, DIM)


# speedup vs baseline: 3.8737x; 1.0477x over previous
import jax
import jax.numpy as jnp
from jax import lax
from jax.experimental import pallas as pl
from jax.experimental.pallas import tpu as pltpu
from jax.experimental.pallas import tpu_sc as plsc

BATCH = 4
SEQ_LEN = 8192
DIM = 1024
NC = 2
NS = 16
NW = NC * NS

EW = SEQ_LEN // NW
CR = 16
CH = CR * DIM
NCH = EW // CR
NSTEP = NCH * BATCH
NB = 5


def _row_off(base, k):
    c, b = k // BATCH, k % BATCH
    return b * SEQ_LEN + base + c * CR


def _sc_body(in_hbm, emb_hbm, out_hbm,
             emb_buf, io_buf, sem_emb, sem_in, sem_out):
    wid = lax.axis_index("s") * NC + lax.axis_index("c")
    base = wid * EW

    def in_cp(k):
        m = k % NB
        return pltpu.make_async_copy(
            in_hbm.at[pl.ds(_row_off(base, k), CR)], io_buf.at[m], sem_in.at[m])

    def out_cp(k):
        m = k % NB
        return pltpu.make_async_copy(
            io_buf.at[m], out_hbm.at[pl.ds(_row_off(base, k), CR)], sem_out.at[m])

    def emb_cp(c):
        q = c % 2
        return pltpu.make_async_copy(
            emb_hbm.at[pl.ds(base + c * CR, CR)], emb_buf.at[q], sem_emb.at[q])

    emb_cp(0).start()
    in_cp(0).start()
    in_cp(1).start()
    in_cp(2).start()

    for k in range(NSTEP):
        m = k % NB
        c, b = k // BATCH, k % BATCH
        q = c % 2
        if b == 0:
            emb_cp(c).wait()
            if c + 1 < NCH:
                emb_cp(c + 1).start()
        in_cp(k).wait()

        @plsc.parallel_loop(0, CH, step=16, unroll=8)
        def _add(i):
            r = i >> 10
            j = pl.multiple_of(i & (DIM - 1), 16)
            io_buf[m, r, pl.ds(j, 16)] += emb_buf[q, r, pl.ds(j, 16)]

        out_cp(k).start()
        if k + 3 < NSTEP:
            if k >= 2:
                out_cp(k - 2).wait()
            in_cp(k + 3).start()

    out_cp(NSTEP - 3).wait()
    out_cp(NSTEP - 2).wait()
    out_cp(NSTEP - 1).wait()


def kernel(inputs, embeddings):
    in2d = inputs.reshape(BATCH * SEQ_LEN, DIM)
    mesh = plsc.VectorSubcoreMesh(core_axis_name="c", subcore_axis_name="s")
    out = pl.kernel(
        _sc_body,
        out_type=jax.ShapeDtypeStruct((BATCH * SEQ_LEN, DIM), jnp.float32),
        mesh=mesh,
        scratch_types=[
            pltpu.VMEM((2, CR, DIM), jnp.float32),
            pltpu.VMEM((NB, CR, DIM), jnp.float32),
            pltpu.SemaphoreType.DMA((2,)),
            pltpu.SemaphoreType.DMA((NB,)),
            pltpu.SemaphoreType.DMA((NB,)),
        ],
    )(in2d, embeddings)
    return out.reshape(BATCH, SEQ_LEN, DIM)
